# Initial kernel scaffold; baseline (speedup 1.0000x reference)
import jax
import jax.numpy as jnp
from jax.experimental import pallas as pl


def kernel(x, table):
    raise NotImplementedError("write your pallas kernel here")


# speedup vs baseline: 7.7665x; 7.7665x over previous
import jax
import jax.numpy as jnp
from jax import lax
from jax.experimental import pallas as pl
from jax.experimental.pallas import tpu as pltpu
from jax.experimental.pallas import tpu_sc as plsc

B = 4096
L = 200
D = 128
NC = 2
NS = 16
NW = NC * NS
BPW = B // NW
C0 = 128
C1 = L - C0
NV = D // 16


def _body(x_hbm, table_hbm, out_hbm, idx_v, rows_v, out_v, sem):
    wid = lax.axis_index("s") * NC + lax.axis_index("c")
    base = wid * BPW

    pltpu.sync_copy(x_hbm.at[pl.ds(base, BPW), :], idx_v)

    def row(r, carry):
        cp0 = pltpu.async_copy(
            table_hbm.at[idx_v.at[r, pl.ds(0, C0)]],
            rows_v.at[pl.ds(0, C0)], sem)
        cp1 = pltpu.async_copy(
            table_hbm.at[idx_v.at[r, pl.ds(C0, C1)]],
            rows_v.at[pl.ds(C0, C1)], sem)
        cp0.wait()
        cp1.wait()

        def acc_body(l, accs):
            return tuple(
                a + rows_v[l, pl.ds(c * 16, 16)] for c, a in enumerate(accs))

        accs = tuple(jnp.zeros((16,), jnp.float32) for _ in range(NV))
        accs = lax.fori_loop(0, L, acc_body, accs)
        for c in range(NV):
            out_v[r, pl.ds(c * 16, 16)] = accs[c]
        return carry

    lax.fori_loop(0, BPW, row, 0)
    pltpu.sync_copy(out_v, out_hbm.at[pl.ds(base, BPW), :])


def kernel(x, table):
    k = pl.kernel(
        _body,
        out_type=jax.ShapeDtypeStruct((B, D), jnp.float32),
        mesh=plsc.VectorSubcoreMesh(core_axis_name="c", subcore_axis_name="s"),
        scratch_types=[
            pltpu.VMEM((BPW, L), jnp.int32),
            pltpu.VMEM((L, D), jnp.float32),
            pltpu.VMEM((BPW, D), jnp.float32),
            pltpu.SemaphoreType.DMA,
        ],
    )
    return k(x, table)


# speedup vs baseline: 13.7562x; 1.7712x over previous
import jax
import jax.numpy as jnp
from jax import lax
from jax.experimental import pallas as pl
from jax.experimental.pallas import tpu as pltpu
from jax.experimental.pallas import tpu_sc as plsc

B = 4096
L = 200
D = 128
NC = 2
NS = 16
NW = NC * NS
BPW = B // NW
C0 = 128
C1 = L - C0
NV = D // 16


def _body(x_hbm, table_hbm, out_hbm, idx_v, rows0, rows1, out_v, sem0, sem1):
    wid = lax.axis_index("s") * NC + lax.axis_index("c")
    base = wid * BPW

    pltpu.sync_copy(x_hbm.at[pl.ds(base, BPW), :], idx_v)

    def start(r, buf, sem):
        pltpu.async_copy(
            table_hbm.at[idx_v.at[r, pl.ds(0, C0)]], buf.at[pl.ds(0, C0)], sem)
        pltpu.async_copy(
            table_hbm.at[idx_v.at[r, pl.ds(C0, C1)]], buf.at[pl.ds(C0, C1)],
            sem)

    def wait(buf, sem):
        pltpu.make_async_copy(
            table_hbm.at[idx_v.at[0, pl.ds(0, C0)]], buf.at[pl.ds(0, C0)],
            sem).wait()
        pltpu.make_async_copy(
            table_hbm.at[idx_v.at[0, pl.ds(C0, C1)]], buf.at[pl.ds(C0, C1)],
            sem).wait()

    def accumulate(r, buf):
        def acc_body(l, accs):
            return tuple(
                a + buf[l, pl.ds(c * 16, 16)] for c, a in enumerate(accs))

        accs = tuple(jnp.zeros((16,), jnp.float32) for _ in range(NV))
        accs = lax.fori_loop(0, L, acc_body, accs)
        for c in range(NV):
            out_v[r, pl.ds(c * 16, 16)] = accs[c]

    bufs = ((rows0, sem0), (rows1, sem1))
    start(0, rows0, sem0)
    start(1, rows1, sem1)

    def pair(i, carry):
        for b, (buf, sem) in enumerate(bufs):
            r = 2 * i + b
            wait(buf, sem)
            accumulate(r, buf)
            start(jnp.minimum(r + 2, BPW - 1), buf, sem)
        return carry

    lax.fori_loop(0, BPW // 2, pair, 0)
    wait(rows0, sem0)
    wait(rows1, sem1)

    pltpu.sync_copy(out_v, out_hbm.at[pl.ds(base, BPW), :])


def kernel(x, table):
    k = pl.kernel(
        _body,
        out_type=jax.ShapeDtypeStruct((B, D), jnp.float32),
        mesh=plsc.VectorSubcoreMesh(core_axis_name="c", subcore_axis_name="s"),
        scratch_types=[
            pltpu.VMEM((BPW, L), jnp.int32),
            pltpu.VMEM((L, D), jnp.float32),
            pltpu.VMEM((L, D), jnp.float32),
            pltpu.VMEM((BPW, D), jnp.float32),
            pltpu.SemaphoreType.DMA,
            pltpu.SemaphoreType.DMA,
        ],
    )
    return k(x, table)


# speedup vs baseline: 15.6644x; 1.1387x over previous
import jax
import jax.numpy as jnp
from jax import lax
from jax.experimental import pallas as pl
from jax.experimental.pallas import tpu as pltpu
from jax.experimental.pallas import tpu_sc as plsc

B = 4096
L = 200
D = 128
NC = 2
NS = 16
NW = NC * NS
BPW = B // NW
U0 = 128
U1 = L - U0
NB = 4
NU = 2 * BPW
NV = D // 16


def _body(x_hbm, table_hbm, out_hbm, idx_v, b0, b1, b2, b3, out_v,
          s0, s1, s2, s3):
    wid = lax.axis_index("s") * NC + lax.axis_index("c")
    base = wid * BPW

    pltpu.sync_copy(x_hbm.at[pl.ds(base, BPW), :], idx_v)

    bufs = ((b0, s0), (b1, s1), (b2, s2), (b3, s3))
    sizes = (U0, U1, U0, U1)
    offs = (0, U0, 0, U0)

    def start(u, b):
        buf, sem = bufs[b]
        r = jnp.minimum(u // 2, BPW - 1)
        pltpu.async_copy(
            table_hbm.at[idx_v.at[r, pl.ds(offs[b], sizes[b])]],
            buf.at[pl.ds(0, sizes[b])], sem)

    def wait(b):
        buf, sem = bufs[b]
        pltpu.make_async_copy(
            table_hbm.at[idx_v.at[0, pl.ds(offs[b], sizes[b])]],
            buf.at[pl.ds(0, sizes[b])], sem).wait()

    def accumulate(buf, n, accs):
        def acc_body(l, accs):
            return tuple(
                a + buf[l, pl.ds(c * 16, 16)] for c, a in enumerate(accs))

        return lax.fori_loop(0, n, acc_body, accs)

    for b in range(NB):
        start(b, b)

    def block(i, carry):
        for half in range(NB // 2):
            r = 2 * i + half
            accs = tuple(jnp.zeros((16,), jnp.float32) for _ in range(NV))
            for p in range(2):
                b = 2 * half + p
                u = 2 * r + p
                wait(b)
                accs = accumulate(bufs[b][0], sizes[b], accs)
                start(u + NB, b)
            for c in range(NV):
                out_v[r, pl.ds(c * 16, 16)] = accs[c]
        return carry

    lax.fori_loop(0, BPW // 2, block, 0)
    for b in range(NB):
        wait(b)

    pltpu.sync_copy(out_v, out_hbm.at[pl.ds(base, BPW), :])


def kernel(x, table):
    k = pl.kernel(
        _body,
        out_type=jax.ShapeDtypeStruct((B, D), jnp.float32),
        mesh=plsc.VectorSubcoreMesh(core_axis_name="c", subcore_axis_name="s"),
        scratch_types=[
            pltpu.VMEM((BPW, L), jnp.int32),
            pltpu.VMEM((U0, D), jnp.float32),
            pltpu.VMEM((U0, D), jnp.float32),
            pltpu.VMEM((U0, D), jnp.float32),
            pltpu.VMEM((U0, D), jnp.float32),
            pltpu.VMEM((BPW, D), jnp.float32),
            pltpu.SemaphoreType.DMA,
            pltpu.SemaphoreType.DMA,
            pltpu.SemaphoreType.DMA,
            pltpu.SemaphoreType.DMA,
        ],
    )
    return k(x, table)
